# baseline (speedup 1.0000x reference)
import functools

import jax
import jax.numpy as jnp
from jax import lax
from jax.experimental import pallas as pl
from jax.experimental.pallas import tpu as pltpu
from jax.experimental.pallas import tpu_sc as plsc


@functools.lru_cache(maxsize=None)
def _build_gather(S, B, V, D, C):
    info = plsc.get_sparse_core_info()
    NC, NS, L = info.num_cores, info.num_subcores, info.num_lanes
    NW = NC * NS
    assert B % (NW * C) == 0 and C % 128 == 0 and D == 32 and L == 16
    CB = C // 128
    DB = D // 8

    mesh = plsc.VectorSubcoreMesh(core_axis_name="c", subcore_axis_name="s")

    @functools.partial(
        pl.kernel,
        mesh=mesh,
        out_type=jax.ShapeDtypeStruct((S, DB, B // 128, 8, 128), jnp.float32),
        scratch_types=[
            pltpu.VMEM((2, C), jnp.int32),
            pltpu.VMEM((2 * C, D), jnp.float32),
            pltpu.VMEM((DB, CB, 8, 128), jnp.float32),
            pltpu.SemaphoreType.DMA((2,)),
            pltpu.SemaphoreType.DMA,
        ],
        compiler_params=pltpu.CompilerParams(
            use_tc_tiling_on_sc=False, needs_layout_passes=False
        ),
    )
    def gather_kernel(idx_hbm, table_hbm, out_hbm, idx_v, rows_v, obuf, gsem, osem):
        wid = lax.axis_index("s") * NC + lax.axis_index("c")
        b0 = wid * C
        lane = lax.iota(jnp.int32, L)

        pltpu.sync_copy(idx_hbm.at[0, pl.ds(b0, C)], idx_v.at[0])
        pltpu.async_copy(
            table_hbm.at[idx_v.at[0]], rows_v.at[pl.ds(0, C)], gsem.at[0]
        )

        def body(s, carry):
            p = lax.rem(s, 2)
            q = 1 - p

            @pl.when(s + 1 < S)
            def _():
                pltpu.sync_copy(idx_hbm.at[s + 1, pl.ds(b0, C)], idx_v.at[q])
                pltpu.async_copy(
                    table_hbm.at[idx_v.at[q]], rows_v.at[pl.ds(q * C, C)], gsem.at[q]
                )

            pltpu.make_async_copy(
                table_hbm.at[idx_v.at[p]], rows_v.at[pl.ds(p * C, C)], gsem.at[p]
            ).wait()

            @pl.when(s > 0)
            def _():
                for dblk in range(DB):
                    pltpu.make_async_copy(
                        obuf.at[dblk],
                        out_hbm.at[0, dblk, pl.ds(wid * CB, CB)],
                        osem,
                    ).wait()

            pbase = p * C

            @plsc.parallel_loop(0, DB * CB, unroll=4)
            def trans(u):
                dblk = u // CB
                bblk = lax.rem(u, CB)
                rbase = pbase + bblk * 128 + lane
                dbase = jnp.full((L,), dblk * 8, jnp.int32)
                for dsub in range(8):
                    dvec = dbase + dsub
                    for grp in range(128 // L):
                        rvec = rbase + grp * L
                        vals = plsc.load_gather(rows_v, [rvec, dvec])
                        obuf[dblk, bblk, dsub, pl.ds(grp * L, L)] = vals

            for dblk in range(DB):
                pltpu.async_copy(
                    obuf.at[dblk],
                    out_hbm.at[s, dblk, pl.ds(wid * CB, CB)],
                    osem,
                )
            return carry

        lax.fori_loop(0, S, body, 0)

        for dblk in range(DB):
            pltpu.make_async_copy(
                obuf.at[dblk],
                out_hbm.at[0, dblk, pl.ds(wid * CB, CB)],
                osem,
            ).wait()

    return gather_kernel


def kernel(inputs, embeddings):
    B, S = inputs.shape
    V, D = embeddings.shape
    idx_t = jnp.transpose(inputs).astype(jnp.int32)
    out5 = _build_gather(S, B, V, D, 512)(idx_t, embeddings)
    out = jnp.transpose(out5, (2, 4, 0, 1, 3)).reshape(B, S, D)
    return out


# speedup vs baseline: 1.0438x; 1.0438x over previous
import functools

import jax
import jax.numpy as jnp
from jax import lax
from jax.experimental import pallas as pl
from jax.experimental.pallas import tpu as pltpu
from jax.experimental.pallas import tpu_sc as plsc


@functools.lru_cache(maxsize=None)
def _build_gather(S, B, V, D, C):
    info = plsc.get_sparse_core_info()
    NC, NS, L = info.num_cores, info.num_subcores, info.num_lanes
    NW = NC * NS
    assert B % (NW * C) == 0 and C % 128 == 0 and D == 32 and L == 16
    CB = C // 128
    DB = D // 8

    mesh = plsc.VectorSubcoreMesh(core_axis_name="c", subcore_axis_name="s")

    @functools.partial(
        pl.kernel,
        mesh=mesh,
        out_type=jax.ShapeDtypeStruct((S, DB, B // 128, 8, 128), jnp.float32),
        scratch_types=[
            pltpu.VMEM((C, S), jnp.int32),
            pltpu.VMEM((2, C), jnp.int32),
            pltpu.VMEM((2 * C, D), jnp.float32),
            pltpu.VMEM((DB, CB, 8, 128), jnp.float32),
            pltpu.SemaphoreType.DMA((2,)),
            pltpu.SemaphoreType.DMA,
        ],
        compiler_params=pltpu.CompilerParams(
            use_tc_tiling_on_sc=False, needs_layout_passes=False
        ),
    )
    def gather_kernel(
        idx_hbm, table_hbm, out_hbm, islab, idx_v, rows_v, obuf, gsem, osem
    ):
        wid = lax.axis_index("s") * NC + lax.axis_index("c")
        b0 = wid * C
        lane = lax.iota(jnp.int32, L)

        pltpu.sync_copy(idx_hbm.at[pl.ds(b0, C)], islab)

        def build_idx(buf, s):
            svec = jnp.full((L,), s, jnp.int32)

            @plsc.parallel_loop(0, C // L, unroll=2)
            def _(k):
                bvec = k * L + lane
                idx_v[buf, pl.ds(k * L, L)] = plsc.load_gather(islab, [bvec, svec])

        build_idx(0, 0)
        pltpu.async_copy(
            table_hbm.at[idx_v.at[0]], rows_v.at[pl.ds(0, C)], gsem.at[0]
        )

        def body(s, carry):
            p = lax.rem(s, 2)
            q = 1 - p

            @pl.when(s + 1 < S)
            def _():
                build_idx(q, s + 1)
                pltpu.async_copy(
                    table_hbm.at[idx_v.at[q]], rows_v.at[pl.ds(q * C, C)], gsem.at[q]
                )

            pltpu.make_async_copy(
                table_hbm.at[idx_v.at[p]], rows_v.at[pl.ds(p * C, C)], gsem.at[p]
            ).wait()

            @pl.when(s > 0)
            def _():
                for dblk in range(DB):
                    pltpu.make_async_copy(
                        obuf.at[dblk],
                        out_hbm.at[0, dblk, pl.ds(wid * CB, CB)],
                        osem,
                    ).wait()

            pbase = p * C

            @plsc.parallel_loop(0, DB * CB, unroll=2)
            def trans(u):
                dblk = u // CB
                bblk = lax.rem(u, CB)
                rbase = pbase + bblk * 128 + lane
                dbase = jnp.full((L,), dblk * 8, jnp.int32)
                for dsub in range(8):
                    dvec = dbase + dsub
                    for grp in range(128 // L):
                        rvec = rbase + grp * L
                        vals = plsc.load_gather(rows_v, [rvec, dvec])
                        obuf[dblk, bblk, dsub, pl.ds(grp * L, L)] = vals

            for dblk in range(DB):
                pltpu.async_copy(
                    obuf.at[dblk],
                    out_hbm.at[s, dblk, pl.ds(wid * CB, CB)],
                    osem,
                )
            return carry

        lax.fori_loop(0, S, body, 0)

        for dblk in range(DB):
            pltpu.make_async_copy(
                obuf.at[dblk],
                out_hbm.at[0, dblk, pl.ds(wid * CB, CB)],
                osem,
            ).wait()

    return gather_kernel


def kernel(inputs, embeddings):
    B, S = inputs.shape
    V, D = embeddings.shape
    out5 = _build_gather(S, B, V, D, 512)(inputs.astype(jnp.int32), embeddings)
    out = jnp.transpose(out5, (2, 4, 0, 1, 3)).reshape(B, S, D)
    return out


# speedup vs baseline: 1.0547x; 1.0105x over previous
import functools

import jax
import jax.numpy as jnp
from jax import lax
from jax.experimental import pallas as pl
from jax.experimental.pallas import tpu as pltpu
from jax.experimental.pallas import tpu_sc as plsc


@functools.lru_cache(maxsize=None)
def _build_gather(S, B, V, D, C):
    info = plsc.get_sparse_core_info()
    NC, NS, L = info.num_cores, info.num_subcores, info.num_lanes
    NW = NC * NS
    assert B % (NW * C) == 0 and C % 128 == 0 and D == 32 and L == 16
    CB = C // 128
    DB = D // 8

    mesh = plsc.VectorSubcoreMesh(core_axis_name="c", subcore_axis_name="s")

    @functools.partial(
        pl.kernel,
        mesh=mesh,
        out_type=jax.ShapeDtypeStruct((S, DB, B // 128, 8, 128), jnp.float32),
        scratch_types=[
            pltpu.VMEM((S, C), jnp.int32),
            pltpu.VMEM((2 * C, D), jnp.float32),
            pltpu.VMEM((DB, CB, 8, 128), jnp.float32),
            pltpu.SemaphoreType.DMA((2,)),
            pltpu.SemaphoreType.DMA,
        ],
        compiler_params=pltpu.CompilerParams(
            use_tc_tiling_on_sc=False, needs_layout_passes=False
        ),
    )
    def gather_kernel(idx_hbm, table_hbm, out_hbm, islab, rows_v, obuf, gsem, osem):
        wid = lax.axis_index("s") * NC + lax.axis_index("c")
        b0 = wid * C
        lane = lax.iota(jnp.int32, L)

        pltpu.sync_copy(idx_hbm.at[pl.ds(0, S), pl.ds(b0, C)], islab)

        pltpu.async_copy(
            table_hbm.at[islab.at[0]], rows_v.at[pl.ds(0, C)], gsem.at[0]
        )

        def body(s, carry):
            p = lax.rem(s, 2)
            q = 1 - p

            @pl.when(s + 1 < S)
            def _():
                pltpu.async_copy(
                    table_hbm.at[islab.at[s + 1]],
                    rows_v.at[pl.ds(q * C, C)],
                    gsem.at[q],
                )

            pltpu.make_async_copy(
                table_hbm.at[islab.at[s]], rows_v.at[pl.ds(p * C, C)], gsem.at[p]
            ).wait()

            @pl.when(s > 0)
            def _():
                for dblk in range(DB):
                    pltpu.make_async_copy(
                        obuf.at[dblk],
                        out_hbm.at[0, dblk, pl.ds(wid * CB, CB)],
                        osem,
                    ).wait()

            pbase = p * C

            @plsc.parallel_loop(0, DB * CB, unroll=2)
            def trans(u):
                dblk = u // CB
                bblk = lax.rem(u, CB)
                rbase = pbase + bblk * 128 + lane
                dbase = jnp.full((L,), dblk * 8, jnp.int32)
                for dsub in range(8):
                    dvec = dbase + dsub
                    for grp in range(128 // L):
                        rvec = rbase + grp * L
                        vals = plsc.load_gather(rows_v, [rvec, dvec])
                        obuf[dblk, bblk, dsub, pl.ds(grp * L, L)] = vals

            for dblk in range(DB):
                pltpu.async_copy(
                    obuf.at[dblk],
                    out_hbm.at[s, dblk, pl.ds(wid * CB, CB)],
                    osem,
                )
            return carry

        lax.fori_loop(0, S, body, 0)

        for dblk in range(DB):
            pltpu.make_async_copy(
                obuf.at[dblk],
                out_hbm.at[0, dblk, pl.ds(wid * CB, CB)],
                osem,
            ).wait()

    return gather_kernel


def kernel(inputs, embeddings):
    B, S = inputs.shape
    V, D = embeddings.shape
    Sp = (S + 7) // 8 * 8
    idx_t = jnp.transpose(
        jnp.pad(inputs.astype(jnp.int32), ((0, 0), (0, Sp - S)))
    )
    out5 = _build_gather(S, B, V, D, 512)(idx_t, embeddings)
    out = jnp.transpose(out5, (2, 4, 0, 1, 3)).reshape(B, S, D)
    return out


# speedup vs baseline: 1.1741x; 1.1131x over previous
import functools

import jax
import jax.numpy as jnp
from jax import lax
from jax.experimental import pallas as pl
from jax.experimental.pallas import tpu as pltpu
from jax.experimental.pallas import tpu_sc as plsc


@functools.lru_cache(maxsize=None)
def _build_gather(S, B, V, D, C):
    info = plsc.get_sparse_core_info()
    NC, NS, L = info.num_cores, info.num_subcores, info.num_lanes
    NW = NC * NS
    assert B % (NW * C) == 0 and C % 128 == 0 and D == 32 and L == 16
    CB = C // 128
    DB = D // 8

    mesh = plsc.VectorSubcoreMesh(core_axis_name="c", subcore_axis_name="s")

    @functools.partial(
        pl.kernel,
        mesh=mesh,
        out_type=jax.ShapeDtypeStruct((S, DB, B // 128, 8, 128), jnp.float32),
        scratch_types=[
            pltpu.VMEM((S, C), jnp.int32),
            pltpu.VMEM((2 * C, D), jnp.float32),
            pltpu.VMEM((DB, CB, 8, 128), jnp.float32),
            pltpu.SemaphoreType.DMA((2,)),
            pltpu.SemaphoreType.DMA,
        ],
        compiler_params=pltpu.CompilerParams(
            use_tc_tiling_on_sc=False, needs_layout_passes=False
        ),
    )
    def gather_kernel(idx_hbm, table_hbm, out_hbm, islab, rows_v, obuf, gsem, osem):
        wid = lax.axis_index("s") * NC + lax.axis_index("c")
        b0 = wid * C
        lane = lax.iota(jnp.int32, L)

        pltpu.sync_copy(idx_hbm.at[pl.ds(0, S), pl.ds(b0, C)], islab)

        pltpu.async_copy(
            table_hbm.at[islab.at[0]], rows_v.at[pl.ds(0, C)], gsem.at[0]
        )

        def body(s, carry):
            p = lax.rem(s, 2)
            q = 1 - p

            @pl.when(s + 1 < S)
            def _():
                pltpu.async_copy(
                    table_hbm.at[islab.at[s + 1]],
                    rows_v.at[pl.ds(q * C, C)],
                    gsem.at[q],
                )

            pltpu.make_async_copy(
                table_hbm.at[islab.at[s]], rows_v.at[pl.ds(p * C, C)], gsem.at[p]
            ).wait()

            @pl.when(s > 0)
            def _():
                for dblk in range(DB):
                    pltpu.make_async_copy(
                        obuf.at[dblk],
                        out_hbm.at[0, dblk, pl.ds(wid * CB, CB)],
                        osem,
                    ).wait()

            pbase = p * C

            @plsc.parallel_loop(0, DB * CB * 8, unroll=4)
            def trans(u):
                dblk = u // (CB * 8)
                rem = lax.rem(u, CB * 8)
                bblk = rem // 8
                dsub = lax.rem(rem, 8)
                rbase = pbase + bblk * 128 + lane
                dvec = jnp.full((L,), dblk * 8 + dsub, jnp.int32)
                for grp in range(128 // L):
                    rvec = rbase + grp * L
                    vals = plsc.load_gather(rows_v, [rvec, dvec])
                    obuf[dblk, bblk, dsub, pl.ds(grp * L, L)] = vals

            for dblk in range(DB):
                pltpu.async_copy(
                    obuf.at[dblk],
                    out_hbm.at[s, dblk, pl.ds(wid * CB, CB)],
                    osem,
                )
            return carry

        lax.fori_loop(0, S, body, 0)

        for dblk in range(DB):
            pltpu.make_async_copy(
                obuf.at[dblk],
                out_hbm.at[0, dblk, pl.ds(wid * CB, CB)],
                osem,
            ).wait()

    return gather_kernel


def kernel(inputs, embeddings):
    B, S = inputs.shape
    V, D = embeddings.shape
    Sp = (S + 7) // 8 * 8
    idx_t = jnp.transpose(
        jnp.pad(inputs.astype(jnp.int32), ((0, 0), (0, Sp - S)))
    )
    out5 = _build_gather(S, B, V, D, 512)(idx_t, embeddings)
    out = jnp.transpose(out5, (2, 4, 0, 1, 3)).reshape(B, S, D)
    return out
